# Initial kernel scaffold; baseline (speedup 1.0000x reference)
import jax
import jax.numpy as jnp
from jax.experimental import pallas as pl


def kernel(en_tokens, hi_tokens, en_table, hi_table):
    raise NotImplementedError("write your pallas kernel here")


# speedup vs baseline: 6.8567x; 6.8567x over previous
import functools

import jax
import jax.numpy as jnp
from jax import lax
from jax.experimental import pallas as pl
from jax.experimental.pallas import tpu as pltpu
from jax.experimental.pallas import tpu_sc as plsc

D_MODEL = 128
CHUNK = 128


@functools.cache
def _build(n_tokens: int, vocab_en: int, vocab_hi: int, d: int):
    info = plsc.get_sparse_core_info()
    nc, ns = info.num_cores, info.num_subcores
    nw = nc * ns
    assert n_tokens % (nw * CHUNK) == 0
    b_per_w = n_tokens // nw
    n_chunks = b_per_w // CHUNK

    mesh = plsc.VectorSubcoreMesh(core_axis_name="c", subcore_axis_name="s")

    @functools.partial(
        pl.kernel,
        out_type=(
            jax.ShapeDtypeStruct((n_tokens, d), jnp.float32),
            jax.ShapeDtypeStruct((n_tokens, d), jnp.float32),
        ),
        mesh=mesh,
        scratch_types=[
            pltpu.VMEM((n_chunks, CHUNK), jnp.int32),
            pltpu.VMEM((CHUNK, d), jnp.float32),
            pltpu.SemaphoreType.DMA,
            pltpu.SemaphoreType.DMA,
        ],
    )
    def k(en_idx, hi_idx, en_table, hi_table, out_en, out_hi,
          idx_v, rows_v, gsem, wsem):
        wid = lax.axis_index("s") * nc + lax.axis_index("c")
        base = wid * b_per_w

        def run_table(idx_hbm, table_hbm, out_hbm):
            pltpu.sync_copy(idx_hbm.at[pl.ds(wid * n_chunks, n_chunks)], idx_v)

            @pl.loop(0, n_chunks)
            def chunk_loop(j):
                pltpu.async_copy(table_hbm.at[idx_v.at[j]], rows_v, gsem).wait()
                pltpu.async_copy(
                    rows_v, out_hbm.at[pl.ds(base + j * CHUNK, CHUNK)], wsem
                ).wait()

        run_table(en_idx, en_table, out_en)
        run_table(hi_idx, hi_table, out_hi)

    return k


def kernel(en_tokens, hi_tokens, en_table, hi_table):
    b, s = en_tokens.shape
    n = b * s
    d = en_table.shape[1]
    k = _build(n, en_table.shape[0], hi_table.shape[0], d)
    en_flat = en_tokens.reshape(n // CHUNK, CHUNK).astype(jnp.int32)
    hi_flat = hi_tokens.reshape(n // CHUNK, CHUNK).astype(jnp.int32)
    out_en, out_hi = k(en_flat, hi_flat, en_table, hi_table)
    return (out_en.reshape(b, s, d), out_hi.reshape(b, s, d))


# speedup vs baseline: 10.0687x; 1.4684x over previous
import functools

import jax
import jax.numpy as jnp
from jax import lax
from jax.experimental import pallas as pl
from jax.experimental.pallas import tpu as pltpu
from jax.experimental.pallas import tpu_sc as plsc

D_MODEL = 128
CHUNK = 128
NBUF = 4


@functools.cache
def _build(n_tokens: int, d: int):
    info = plsc.get_sparse_core_info()
    nc, ns = info.num_cores, info.num_subcores
    nw = nc * ns
    assert n_tokens % (nw * CHUNK) == 0
    b_per_w = n_tokens // nw
    n_chunks = b_per_w // CHUNK
    assert n_chunks % NBUF == 0 and n_chunks >= 2 * NBUF

    mesh = plsc.VectorSubcoreMesh(core_axis_name="c", subcore_axis_name="s")

    @functools.partial(
        pl.kernel,
        out_type=(
            jax.ShapeDtypeStruct((n_tokens, d), jnp.float32),
            jax.ShapeDtypeStruct((n_tokens, d), jnp.float32),
        ),
        mesh=mesh,
        scratch_types=[
            pltpu.VMEM((n_chunks, CHUNK), jnp.int32),
            [pltpu.VMEM((CHUNK, d), jnp.float32)] * NBUF,
            [pltpu.SemaphoreType.DMA] * NBUF,
            [pltpu.SemaphoreType.DMA] * NBUF,
        ],
    )
    def k(en_idx, hi_idx, en_table, hi_table, out_en, out_hi,
          idx_v, rows, gsem, wsem):
        wid = lax.axis_index("s") * nc + lax.axis_index("c")
        base = wid * b_per_w

        def run_table(idx_hbm, table_hbm, out_hbm):
            pltpu.sync_copy(idx_hbm.at[pl.ds(wid * n_chunks, n_chunks)], idx_v)

            def gather(j, b):
                pltpu.async_copy(table_hbm.at[idx_v.at[j]], rows[b], gsem[b])

            def gather_wait(j, b):
                pltpu.make_async_copy(
                    table_hbm.at[idx_v.at[j]], rows[b], gsem[b]
                ).wait()

            def wb(j, b):
                pltpu.async_copy(
                    rows[b], out_hbm.at[pl.ds(base + j * CHUNK, CHUNK)], wsem[b]
                )

            def wb_wait(j, b):
                pltpu.make_async_copy(
                    rows[b], out_hbm.at[pl.ds(base + j * CHUNK, CHUNK)], wsem[b]
                ).wait()

            for b in range(NBUF):
                gather(b, b)

            @pl.loop(0, n_chunks - NBUF, step=NBUF)
            def body(g):
                for b in range(NBUF):
                    gather_wait(g + b, b)
                    wb(g + b, b)
                for b in range(NBUF):
                    wb_wait(g + b, b)
                    gather(g + NBUF + b, b)

            for b in range(NBUF):
                j = n_chunks - NBUF + b
                gather_wait(j, b)
                wb(j, b)
            for b in range(NBUF):
                j = n_chunks - NBUF + b
                wb_wait(j, b)

        run_table(en_idx, en_table, out_en)
        run_table(hi_idx, hi_table, out_hi)

    return k


def kernel(en_tokens, hi_tokens, en_table, hi_table):
    b, s = en_tokens.shape
    n = b * s
    d = en_table.shape[1]
    k = _build(n, d)
    en_flat = en_tokens.reshape(n // CHUNK, CHUNK).astype(jnp.int32)
    hi_flat = hi_tokens.reshape(n // CHUNK, CHUNK).astype(jnp.int32)
    out_en, out_hi = k(en_flat, hi_flat, en_table, hi_table)
    return (out_en.reshape(b, s, d), out_hi.reshape(b, s, d))
